# Initial kernel scaffold; baseline (speedup 1.0000x reference)
import jax
import jax.numpy as jnp
from jax.experimental import pallas as pl


def kernel(x_num, W_gate, b_gate, value_proj, feature_embed, emb_tables, Wb1, bb1, Wb2, bb2, Wbase, bbase, ln1_g, ln1_b, Wq, bq, Wk, bk, Wv, bv, Wo, bo, ln2_g, ln2_b, Wf1, bf1, Wf2, bf2, lnd_g, lnd_b, Wd1, bd1, Wd2, bd2, lnb_g, lnb_b, Wbe1, bbe1, Wbe2, bbe2, x_cat):
    raise NotImplementedError("write your pallas kernel here")


# speedup vs baseline: 1.7307x; 1.7307x over previous
import functools
import math

import jax
import jax.numpy as jnp
from jax.experimental import pallas as pl
from jax.experimental.pallas import tpu as pltpu

B = 16384
D = 256
NCAT = 8
CARD = 100
EMB = 32
TOK = 64
NH = 4
HD = TOK // NH
K = 64
DTOK = 256
HID = 512
FF = 128
DELTA_IN = DTOK + TOK
DH = 80
BETA_IN = DTOK + 4
BH = 64

R = 128


def _ln(x, g, b):
    m = jnp.mean(x, axis=-1, keepdims=True)
    v = jnp.mean((x - m) * (x - m), axis=-1, keepdims=True)
    return (x - m) * jax.lax.rsqrt(v + 1e-5) * g + b


def _fwd_body(x_ref, xcat_ref, Wg_ref, bg_ref, VF_ref, EMBBD_ref,
              Wb1_ref, bb1_ref, Wb2_ref, bb2_ref, Wbase_ref, bbase_ref,
              ln1g_ref, ln1b_ref, Wqkv_ref, bqkv_ref, Wo_ref, bo_ref,
              ln2g_ref, ln2b_ref, Wf1_ref, bf1_ref, Wf2_ref, bf2_ref,
              lndg_ref, lndb_ref, Wd1_ref, bd1_ref, Wd2_ref, bd2_ref,
              lnbg_ref, lnbb_ref, Wbe1_ref, bbe1_ref, Wbe2_ref, bbe2_ref,
              out_ref, q_s, k_s, v_s, ao_s, tg_s):
    f32 = jnp.float32
    x = x_ref[...]
    gate_logit = jnp.dot(x, Wg_ref[...], preferred_element_type=f32) + bg_ref[...][None, :]
    gate = jax.nn.sigmoid(gate_logit)
    x_gated = x * gate

    xcat = xcat_ref[...]
    offs = jax.lax.broadcasted_iota(jnp.int32, (1, NCAT), 1) * CARD
    catg = xcat + offs
    i800 = jax.lax.broadcasted_iota(jnp.int32, (R, NCAT * CARD), 1)
    onehot = jnp.zeros((R, NCAT * CARD), f32)
    for f in range(NCAT):
        onehot = onehot + (i800 == catg[:, f:f + 1]).astype(f32)
    cat_feats = jnp.dot(onehot, EMBBD_ref[...], preferred_element_type=f32)

    xin = jnp.concatenate([x_gated, cat_feats], axis=1)
    h = jax.nn.relu(jnp.dot(xin, Wb1_ref[...], preferred_element_type=f32) + bb1_ref[...][None, :])
    h_base = jnp.dot(h, Wb2_ref[...], preferred_element_type=f32) + bb2_ref[...][None, :]
    y_base = jnp.sum(h_base * Wbase_ref[...][:, 0][None, :], axis=1, keepdims=True) + bbase_ref[...][None, :]

    iota_d = jax.lax.broadcasted_iota(jnp.int32, (R, D), 1)
    iota_k = jax.lax.broadcasted_iota(jnp.int32, (1, K), 1)

    def peel(t, carry):
        g_work, tg, ti = carry
        m = jnp.max(g_work, axis=1, keepdims=True)
        hit = g_work == m
        idx = jnp.min(jnp.where(hit, iota_d, D), axis=1, keepdims=True)
        g_work = jnp.where(iota_d == idx, -1.0, g_work)
        slot = iota_k == t
        tg = tg + jnp.where(slot, m, 0.0)
        ti = ti + jnp.where(slot, idx, 0)
        return g_work, tg, ti

    _, topk_g, topk_i = jax.lax.fori_loop(
        0, K, peel, (gate, jnp.zeros((R, K), f32), jnp.zeros((R, K), jnp.int32)))

    topk_w = topk_g / (jnp.sum(topk_g, axis=1, keepdims=True) + 1e-6)

    iota3 = jax.lax.broadcasted_iota(jnp.int32, (R, K, D), 2)
    S = (iota3 == topk_i[:, :, None]).astype(f32)
    topk_x = jnp.sum(S * x[:, None, :], axis=2)
    vf = jnp.dot(S.reshape(R * K, D), VF_ref[...], preferred_element_type=f32)
    vf3 = vf.reshape(R, K, 2 * TOK)
    vp = vf3[:, :, :TOK]
    fe = vf3[:, :, TOK:]
    tokens = (topk_x[:, :, None] * vp + fe) * topk_w[:, :, None]

    res = tokens
    x1 = _ln(tokens, ln1g_ref[...], ln1b_ref[...])
    xf = x1.reshape(R * K, TOK)
    qkv = jnp.dot(xf, Wqkv_ref[...], preferred_element_type=f32) + bqkv_ref[...][None, :]
    q = qkv[:, :TOK].reshape(R, K, TOK)
    k = qkv[:, TOK:2 * TOK].reshape(R, K, TOK)
    v = qkv[:, 2 * TOK:].reshape(R, K, TOK)

    inv_sqrt_hd = 1.0 / math.sqrt(HD)
    CH = 8
    q_s[...] = q
    k_s[...] = k
    v_s[...] = v
    tg_s[...] = topk_g

    def att_chunk(c, dummy):
        r0 = c * CH
        qc = q_s[pl.ds(r0, CH)]
        kc = k_s[pl.ds(r0, CH)]
        vc = v_s[pl.ds(r0, CH)]
        bias_c = tg_s[pl.ds(r0, CH)][:, None, :]
        ao_heads = []
        for hh in range(NH):
            sl = slice(hh * HD, (hh + 1) * HD)
            qh = qc[:, :, sl]
            kh = kc[:, :, sl]
            vh = vc[:, :, sl]
            sc = jax.lax.dot_general(qh, kh, (((2,), (2,)), ((0,), (0,))),
                                     preferred_element_type=f32) * inv_sqrt_hd
            sc = sc + bias_c
            mx = jnp.max(sc, axis=2, keepdims=True)
            e = jnp.exp(sc - mx)
            attn = e / jnp.sum(e, axis=2, keepdims=True)
            ao_h = jax.lax.dot_general(attn, vh, (((2,), (1,)), ((0,), (0,))),
                                       preferred_element_type=f32)
            ao_heads.append(ao_h)
        ao_c = jnp.concatenate(ao_heads, axis=2)
        ao_s[pl.ds(r0, CH)] = ao_c
        return dummy

    jax.lax.fori_loop(0, R // CH, att_chunk, 0)
    ao = ao_s[...]
    xo = res + (jnp.dot(ao.reshape(R * K, TOK), Wo_ref[...], preferred_element_type=f32)
                + bo_ref[...][None, :]).reshape(R, K, TOK)
    x2 = _ln(xo, ln2g_ref[...], ln2b_ref[...])
    ffh = jnp.dot(x2.reshape(R * K, TOK), Wf1_ref[...], preferred_element_type=f32) + bf1_ref[...][None, :]
    ffh = 0.5 * ffh * (1.0 + jax.lax.erf(ffh * (1.0 / math.sqrt(2.0))))
    ffo = jnp.dot(ffh, Wf2_ref[...], preferred_element_type=f32) + bf2_ref[...][None, :]
    xo = xo + ffo.reshape(R, K, TOK)
    z_int = jnp.mean(xo, axis=1)

    dh_in = jnp.concatenate([h_base, z_int], axis=1)
    dh = _ln(dh_in, lndg_ref[...], lndb_ref[...])
    d1 = jax.nn.relu(jnp.dot(dh, Wd1_ref[...], preferred_element_type=f32) + bd1_ref[...][None, :])
    delta = jnp.sum(d1 * Wd2_ref[...][:, 0][None, :], axis=1, keepdims=True) + bd2_ref[...][None, :]

    gm = jnp.mean(gate, axis=1, keepdims=True)
    gx = jnp.max(gate, axis=1, keepdims=True)
    gs = jnp.sqrt(jnp.mean((gate - gm) * (gate - gm), axis=1, keepdims=True))
    bh_in = jnp.concatenate([h_base, y_base, gm, gx, gs], axis=1)
    bh = _ln(bh_in, lnbg_ref[...], lnbb_ref[...])
    b1 = jax.nn.relu(jnp.dot(bh, Wbe1_ref[...], preferred_element_type=f32) + bbe1_ref[...][None, :])
    beta = jax.nn.sigmoid(jnp.sum(b1 * Wbe2_ref[...][:, 0][None, :], axis=1, keepdims=True)
                          + bbe2_ref[...][None, :])

    out_ref[...] = (y_base + beta * delta)[:, 0]


def kernel(x_num, W_gate, b_gate, value_proj, feature_embed, emb_tables, Wb1, bb1, Wb2, bb2, Wbase, bbase, ln1_g, ln1_b, Wq, bq, Wk, bk, Wv, bv, Wo, bo, ln2_g, ln2_b, Wf1, bf1, Wf2, bf2, lnd_g, lnd_b, Wd1, bd1, Wd2, bd2, lnb_g, lnb_b, Wbe1, bbe1, Wbe2, bbe2, x_cat):
    emb_bd = jnp.zeros((NCAT * CARD, NCAT * EMB), jnp.float32)
    for f in range(NCAT):
        emb_bd = emb_bd.at[f * CARD:(f + 1) * CARD, f * EMB:(f + 1) * EMB].set(emb_tables[f])
    VF = jnp.concatenate([value_proj, feature_embed], axis=1)
    Wqkv = jnp.concatenate([Wq, Wk, Wv], axis=1)
    bqkv = jnp.concatenate([bq, bk, bv], axis=0)
    x_cat = x_cat.astype(jnp.int32)

    n_tiles = B // R

    def full2(a):
        return pl.BlockSpec(a.shape, lambda i: (0, 0))

    def full1(a):
        return pl.BlockSpec(a.shape, lambda i: (0,))

    in_specs = [
        pl.BlockSpec((R, D), lambda i: (i, 0)),
        pl.BlockSpec((R, NCAT), lambda i: (i, 0)),
        full2(W_gate), full1(b_gate), full2(VF), full2(emb_bd),
        full2(Wb1), full1(bb1), full2(Wb2), full1(bb2), full2(Wbase), full1(bbase),
        full1(ln1_g), full1(ln1_b), full2(Wqkv), full1(bqkv), full2(Wo), full1(bo),
        full1(ln2_g), full1(ln2_b), full2(Wf1), full1(bf1), full2(Wf2), full1(bf2),
        full1(lnd_g), full1(lnd_b), full2(Wd1), full1(bd1), full2(Wd2), full1(bd2),
        full1(lnb_g), full1(lnb_b), full2(Wbe1), full1(bbe1), full2(Wbe2), full1(bbe2),
    ]

    out = pl.pallas_call(
        _fwd_body,
        grid=(n_tiles,),
        in_specs=in_specs,
        out_specs=pl.BlockSpec((R,), lambda i: (i,)),
        out_shape=jax.ShapeDtypeStruct((B,), jnp.float32),
        scratch_shapes=[
            pltpu.VMEM((R, K, TOK), jnp.float32),
            pltpu.VMEM((R, K, TOK), jnp.float32),
            pltpu.VMEM((R, K, TOK), jnp.float32),
            pltpu.VMEM((R, K, TOK), jnp.float32),
            pltpu.VMEM((R, K), jnp.float32),
        ],
    )(x_num, x_cat, W_gate, b_gate, VF, emb_bd,
      Wb1, bb1, Wb2, bb2, Wbase, bbase,
      ln1_g, ln1_b, Wqkv, bqkv, Wo, bo,
      ln2_g, ln2_b, Wf1, bf1, Wf2, bf2,
      lnd_g, lnd_b, Wd1, bd1, Wd2, bd2,
      lnb_g, lnb_b, Wbe1, bbe1, Wbe2, bbe2)
    return out


# speedup vs baseline: 1.7317x; 1.0005x over previous
import functools
import math

import jax
import jax.numpy as jnp
from jax.experimental import pallas as pl
from jax.experimental.pallas import tpu as pltpu

B = 16384
D = 256
NCAT = 8
CARD = 100
EMB = 32
TOK = 64
NH = 4
HD = TOK // NH
K = 64
DTOK = 256
HID = 512
FF = 128
DELTA_IN = DTOK + TOK
DH = 80
BETA_IN = DTOK + 4
BH = 64

R = 128


def _ln(x, g, b):
    m = jnp.mean(x, axis=-1, keepdims=True)
    v = jnp.mean((x - m) * (x - m), axis=-1, keepdims=True)
    return (x - m) * jax.lax.rsqrt(v + 1e-5) * g + b


def _fwd_body(x_ref, xcat_ref, Wg_ref, bg_ref, VF_ref, EMBBD_ref,
              Wb1_ref, bb1_ref, Wb2_ref, bb2_ref, Wbase_ref, bbase_ref,
              ln1g_ref, ln1b_ref, Wqkv_ref, bqkv_ref, Wo_ref, bo_ref,
              ln2g_ref, ln2b_ref, Wf1_ref, bf1_ref, Wf2_ref, bf2_ref,
              lndg_ref, lndb_ref, Wd1_ref, bd1_ref, Wd2_ref, bd2_ref,
              lnbg_ref, lnbb_ref, Wbe1_ref, bbe1_ref, Wbe2_ref, bbe2_ref,
              out_ref, q_s, k_s, v_s, ao_s, tg_s):
    f32 = jnp.float32
    x = x_ref[...]
    gate_logit = jnp.dot(x, Wg_ref[...], preferred_element_type=f32) + bg_ref[...][None, :]
    gate = jax.nn.sigmoid(gate_logit)
    x_gated = x * gate

    xcat = xcat_ref[...]
    offs = jax.lax.broadcasted_iota(jnp.int32, (1, NCAT), 1) * CARD
    catg = xcat + offs
    i800 = jax.lax.broadcasted_iota(jnp.int32, (R, NCAT * CARD), 1)
    onehot = jnp.zeros((R, NCAT * CARD), f32)
    for f in range(NCAT):
        onehot = onehot + (i800 == catg[:, f:f + 1]).astype(f32)
    cat_feats = jnp.dot(onehot, EMBBD_ref[...], preferred_element_type=f32)

    xin = jnp.concatenate([x_gated, cat_feats], axis=1)
    h = jax.nn.relu(jnp.dot(xin, Wb1_ref[...], preferred_element_type=f32) + bb1_ref[...][None, :])
    h_base = jnp.dot(h, Wb2_ref[...], preferred_element_type=f32) + bb2_ref[...][None, :]
    y_base = jnp.sum(h_base * Wbase_ref[...][:, 0][None, :], axis=1, keepdims=True) + bbase_ref[...][None, :]

    iota_d = jax.lax.broadcasted_iota(jnp.int32, (R, D), 1)
    iota_k = jax.lax.broadcasted_iota(jnp.int32, (1, K), 1)

    def peel(t, carry):
        g_work, tg, ti = carry
        m = jnp.max(g_work, axis=1, keepdims=True)
        hit = g_work == m
        idx = jnp.min(jnp.where(hit, iota_d, D), axis=1, keepdims=True)
        g_work = jnp.where(iota_d == idx, -1.0, g_work)
        slot = iota_k == t
        tg = tg + jnp.where(slot, m, 0.0)
        ti = ti + jnp.where(slot, idx, 0)
        return g_work, tg, ti

    _, topk_g, topk_i = jax.lax.fori_loop(
        0, K, peel, (gate, jnp.zeros((R, K), f32), jnp.zeros((R, K), jnp.int32)))

    topk_w = topk_g / (jnp.sum(topk_g, axis=1, keepdims=True) + 1e-6)

    iota3 = jax.lax.broadcasted_iota(jnp.int32, (R, K, D), 2)
    S = (iota3 == topk_i[:, :, None]).astype(f32)
    topk_x = jnp.sum(S * x[:, None, :], axis=2)
    Sb = S.reshape(R * K, D).astype(jnp.bfloat16)
    VFf = VF_ref[...]
    VF_hi = VFf.astype(jnp.bfloat16)
    VF_lo = (VFf - VF_hi.astype(f32)).astype(jnp.bfloat16)
    vf = (jnp.dot(Sb, jnp.concatenate([VF_hi, VF_lo], axis=1),
                  preferred_element_type=f32))
    vf = vf[:, :2 * TOK] + vf[:, 2 * TOK:]
    vf3 = vf.reshape(R, K, 2 * TOK)
    vp = vf3[:, :, :TOK]
    fe = vf3[:, :, TOK:]
    tokens = (topk_x[:, :, None] * vp + fe) * topk_w[:, :, None]

    res = tokens
    x1 = _ln(tokens, ln1g_ref[...], ln1b_ref[...])
    xf = x1.reshape(R * K, TOK)
    qkv = jnp.dot(xf, Wqkv_ref[...], preferred_element_type=f32) + bqkv_ref[...][None, :]
    q = qkv[:, :TOK].reshape(R, K, TOK)
    k = qkv[:, TOK:2 * TOK].reshape(R, K, TOK)
    v = qkv[:, 2 * TOK:].reshape(R, K, TOK)

    inv_sqrt_hd = 1.0 / math.sqrt(HD)
    CH = 8
    q_s[...] = q
    k_s[...] = k
    v_s[...] = v
    tg_s[...] = topk_g

    def att_chunk(c, dummy):
        r0 = c * CH
        qc = q_s[pl.ds(r0, CH)]
        kc = k_s[pl.ds(r0, CH)]
        vc = v_s[pl.ds(r0, CH)]
        bias_c = tg_s[pl.ds(r0, CH)][:, None, :]
        ao_heads = []
        for hh in range(NH):
            sl = slice(hh * HD, (hh + 1) * HD)
            qh = qc[:, :, sl]
            kh = kc[:, :, sl]
            vh = vc[:, :, sl]
            sc = jax.lax.dot_general(qh, kh, (((2,), (2,)), ((0,), (0,))),
                                     preferred_element_type=f32) * inv_sqrt_hd
            sc = sc + bias_c
            mx = jnp.max(sc, axis=2, keepdims=True)
            e = jnp.exp(sc - mx)
            attn = e / jnp.sum(e, axis=2, keepdims=True)
            ao_h = jax.lax.dot_general(attn, vh, (((2,), (1,)), ((0,), (0,))),
                                       preferred_element_type=f32)
            ao_heads.append(ao_h)
        ao_c = jnp.concatenate(ao_heads, axis=2)
        ao_s[pl.ds(r0, CH)] = ao_c
        return dummy

    jax.lax.fori_loop(0, R // CH, att_chunk, 0)
    ao = ao_s[...]
    xo = res + (jnp.dot(ao.reshape(R * K, TOK), Wo_ref[...], preferred_element_type=f32)
                + bo_ref[...][None, :]).reshape(R, K, TOK)
    x2 = _ln(xo, ln2g_ref[...], ln2b_ref[...])
    ffh = jnp.dot(x2.reshape(R * K, TOK), Wf1_ref[...], preferred_element_type=f32) + bf1_ref[...][None, :]
    ffh = 0.5 * ffh * (1.0 + jax.lax.erf(ffh * (1.0 / math.sqrt(2.0))))
    ffo = jnp.dot(ffh, Wf2_ref[...], preferred_element_type=f32) + bf2_ref[...][None, :]
    xo = xo + ffo.reshape(R, K, TOK)
    z_int = jnp.mean(xo, axis=1)

    dh_in = jnp.concatenate([h_base, z_int], axis=1)
    dh = _ln(dh_in, lndg_ref[...], lndb_ref[...])
    d1 = jax.nn.relu(jnp.dot(dh, Wd1_ref[...], preferred_element_type=f32) + bd1_ref[...][None, :])
    delta = jnp.sum(d1 * Wd2_ref[...][:, 0][None, :], axis=1, keepdims=True) + bd2_ref[...][None, :]

    gm = jnp.mean(gate, axis=1, keepdims=True)
    gx = jnp.max(gate, axis=1, keepdims=True)
    gs = jnp.sqrt(jnp.mean((gate - gm) * (gate - gm), axis=1, keepdims=True))
    bh_in = jnp.concatenate([h_base, y_base, gm, gx, gs], axis=1)
    bh = _ln(bh_in, lnbg_ref[...], lnbb_ref[...])
    b1 = jax.nn.relu(jnp.dot(bh, Wbe1_ref[...], preferred_element_type=f32) + bbe1_ref[...][None, :])
    beta = jax.nn.sigmoid(jnp.sum(b1 * Wbe2_ref[...][:, 0][None, :], axis=1, keepdims=True)
                          + bbe2_ref[...][None, :])

    out_ref[...] = (y_base + beta * delta)[:, 0]


def kernel(x_num, W_gate, b_gate, value_proj, feature_embed, emb_tables, Wb1, bb1, Wb2, bb2, Wbase, bbase, ln1_g, ln1_b, Wq, bq, Wk, bk, Wv, bv, Wo, bo, ln2_g, ln2_b, Wf1, bf1, Wf2, bf2, lnd_g, lnd_b, Wd1, bd1, Wd2, bd2, lnb_g, lnb_b, Wbe1, bbe1, Wbe2, bbe2, x_cat):
    emb_bd = jnp.zeros((NCAT * CARD, NCAT * EMB), jnp.float32)
    for f in range(NCAT):
        emb_bd = emb_bd.at[f * CARD:(f + 1) * CARD, f * EMB:(f + 1) * EMB].set(emb_tables[f])
    VF = jnp.concatenate([value_proj, feature_embed], axis=1)
    Wqkv = jnp.concatenate([Wq, Wk, Wv], axis=1)
    bqkv = jnp.concatenate([bq, bk, bv], axis=0)
    x_cat = x_cat.astype(jnp.int32)

    n_tiles = B // R

    def full2(a):
        return pl.BlockSpec(a.shape, lambda i: (0, 0))

    def full1(a):
        return pl.BlockSpec(a.shape, lambda i: (0,))

    in_specs = [
        pl.BlockSpec((R, D), lambda i: (i, 0)),
        pl.BlockSpec((R, NCAT), lambda i: (i, 0)),
        full2(W_gate), full1(b_gate), full2(VF), full2(emb_bd),
        full2(Wb1), full1(bb1), full2(Wb2), full1(bb2), full2(Wbase), full1(bbase),
        full1(ln1_g), full1(ln1_b), full2(Wqkv), full1(bqkv), full2(Wo), full1(bo),
        full1(ln2_g), full1(ln2_b), full2(Wf1), full1(bf1), full2(Wf2), full1(bf2),
        full1(lnd_g), full1(lnd_b), full2(Wd1), full1(bd1), full2(Wd2), full1(bd2),
        full1(lnb_g), full1(lnb_b), full2(Wbe1), full1(bbe1), full2(Wbe2), full1(bbe2),
    ]

    out = pl.pallas_call(
        _fwd_body,
        grid=(n_tiles,),
        in_specs=in_specs,
        out_specs=pl.BlockSpec((R,), lambda i: (i,)),
        out_shape=jax.ShapeDtypeStruct((B,), jnp.float32),
        compiler_params=pltpu.CompilerParams(
            dimension_semantics=("parallel",)),
        scratch_shapes=[
            pltpu.VMEM((R, K, TOK), jnp.float32),
            pltpu.VMEM((R, K, TOK), jnp.float32),
            pltpu.VMEM((R, K, TOK), jnp.float32),
            pltpu.VMEM((R, K, TOK), jnp.float32),
            pltpu.VMEM((R, K), jnp.float32),
        ],
    )(x_num, x_cat, W_gate, b_gate, VF, emb_bd,
      Wb1, bb1, Wb2, bb2, Wbase, bbase,
      ln1_g, ln1_b, Wqkv, bqkv, Wo, bo,
      ln2_g, ln2_b, Wf1, bf1, Wf2, bf2,
      lnd_g, lnd_b, Wd1, bd1, Wd2, bd2,
      lnb_g, lnb_b, Wbe1, bbe1, Wbe2, bbe2)
    return out


# speedup vs baseline: 2.1729x; 1.2548x over previous
import functools
import math

import jax
import jax.numpy as jnp
from jax.experimental import pallas as pl
from jax.experimental.pallas import tpu as pltpu

B = 16384
D = 256
NCAT = 8
CARD = 100
EMB = 32
TOK = 64
NH = 4
HD = TOK // NH
K = 64
DTOK = 256
HID = 512
FF = 128
DELTA_IN = DTOK + TOK
DH = 80
BETA_IN = DTOK + 4
BH = 64

R = 128


def _ln(x, g, b):
    m = jnp.mean(x, axis=-1, keepdims=True)
    v = jnp.mean((x - m) * (x - m), axis=-1, keepdims=True)
    return (x - m) * jax.lax.rsqrt(v + 1e-5) * g + b


def _fwd_body(x_ref, xcat_ref, Wg_ref, bg_ref, VF_ref, EMBBD_ref, LT2_ref,
              Wb1_ref, bb1_ref, Wb2_ref, bb2_ref, Wbase_ref, bbase_ref,
              ln1g_ref, ln1b_ref, Wqkv_ref, bqkv_ref, Wo_ref, bo_ref,
              ln2g_ref, ln2b_ref, Wf1_ref, bf1_ref, Wf2_ref, bf2_ref,
              lndg_ref, lndb_ref, Wd1_ref, bd1_ref, Wd2_ref, bd2_ref,
              lnbg_ref, lnbb_ref, Wbe1_ref, bbe1_ref, Wbe2_ref, bbe2_ref,
              out_ref, q_s, k_s, v_s, ao_s, tg_s):
    f32 = jnp.float32
    x = x_ref[...]
    gate_logit = jnp.dot(x, Wg_ref[...], preferred_element_type=f32) + bg_ref[...][None, :]
    gate = jax.nn.sigmoid(gate_logit)
    x_gated = x * gate

    xcat = xcat_ref[...]
    offs = jax.lax.broadcasted_iota(jnp.int32, (1, NCAT), 1) * CARD
    catg = xcat + offs
    i800 = jax.lax.broadcasted_iota(jnp.int32, (R, NCAT * CARD), 1)
    onehot = jnp.zeros((R, NCAT * CARD), f32)
    for f in range(NCAT):
        onehot = onehot + (i800 == catg[:, f:f + 1]).astype(f32)
    cat_feats = jnp.dot(onehot, EMBBD_ref[...], preferred_element_type=f32)

    xin = jnp.concatenate([x_gated, cat_feats], axis=1)
    h = jax.nn.relu(jnp.dot(xin, Wb1_ref[...], preferred_element_type=f32) + bb1_ref[...][None, :])
    h_base = jnp.dot(h, Wb2_ref[...], preferred_element_type=f32) + bb2_ref[...][None, :]
    y_base = jnp.sum(h_base * Wbase_ref[...][:, 0][None, :], axis=1, keepdims=True) + bbase_ref[...][None, :]

    def bisect(t, carry):
        lo, hi = carry
        mid = 0.5 * (lo + hi)
        c = jnp.sum((gate > mid).astype(f32), axis=1, keepdims=True)
        geK = c >= K
        return jnp.where(geK, mid, lo), jnp.where(geK, hi, mid)

    lo0 = jnp.zeros((R, 1), f32)
    hi0 = jnp.max(gate, axis=1, keepdims=True)
    lo, hi = jax.lax.fori_loop(0, 48, bisect, (lo0, hi0))

    strict = (gate > hi).astype(f32)
    eq = (gate == hi).astype(f32)
    CC = jnp.concatenate([strict, eq], axis=1).astype(jnp.bfloat16)
    cums = jnp.dot(CC, LT2_ref[...], preferred_element_type=f32)
    csel = cums[:, :D]
    ceq = cums[:, D:]
    c1 = jnp.sum(strict, axis=1, keepdims=True)
    need = K - c1
    eq_take = eq * (ceq < need).astype(f32)
    sel = strict + eq_take
    slot = (csel + jnp.minimum(ceq, need)).astype(jnp.int32)

    iota_k3 = jax.lax.broadcasted_iota(jnp.int32, (1, K, 1), 1)
    S = (slot[:, None, :] == iota_k3).astype(f32) * sel[:, None, :]
    topk_g = jnp.sum(S * gate[:, None, :], axis=2)
    topk_w = topk_g / (jnp.sum(topk_g, axis=1, keepdims=True) + 1e-6)
    topk_x = jnp.sum(S * x[:, None, :], axis=2)
    Sb = S.reshape(R * K, D).astype(jnp.bfloat16)
    VFf = VF_ref[...]
    VF_hi = VFf.astype(jnp.bfloat16)
    VF_lo = (VFf - VF_hi.astype(f32)).astype(jnp.bfloat16)
    vf = (jnp.dot(Sb, jnp.concatenate([VF_hi, VF_lo], axis=1),
                  preferred_element_type=f32))
    vf = vf[:, :2 * TOK] + vf[:, 2 * TOK:]
    vf3 = vf.reshape(R, K, 2 * TOK)
    vp = vf3[:, :, :TOK]
    fe = vf3[:, :, TOK:]
    tokens = (topk_x[:, :, None] * vp + fe) * topk_w[:, :, None]

    res = tokens
    x1 = _ln(tokens, ln1g_ref[...], ln1b_ref[...])
    xf = x1.reshape(R * K, TOK)
    qkv = jnp.dot(xf, Wqkv_ref[...], preferred_element_type=f32) + bqkv_ref[...][None, :]
    q = qkv[:, :TOK].reshape(R, K, TOK)
    k = qkv[:, TOK:2 * TOK].reshape(R, K, TOK)
    v = qkv[:, 2 * TOK:].reshape(R, K, TOK)

    inv_sqrt_hd = 1.0 / math.sqrt(HD)
    CH = 8
    q_s[...] = q
    k_s[...] = k
    v_s[...] = v
    tg_s[...] = topk_g

    def att_chunk(c, dummy):
        r0 = c * CH
        qc = q_s[pl.ds(r0, CH)]
        kc = k_s[pl.ds(r0, CH)]
        vc = v_s[pl.ds(r0, CH)]
        bias_c = tg_s[pl.ds(r0, CH)][:, None, :]
        ao_heads = []
        for hh in range(NH):
            sl = slice(hh * HD, (hh + 1) * HD)
            qh = qc[:, :, sl]
            kh = kc[:, :, sl]
            vh = vc[:, :, sl]
            sc = jax.lax.dot_general(qh, kh, (((2,), (2,)), ((0,), (0,))),
                                     preferred_element_type=f32) * inv_sqrt_hd
            sc = sc + bias_c
            mx = jnp.max(sc, axis=2, keepdims=True)
            e = jnp.exp(sc - mx)
            attn = e / jnp.sum(e, axis=2, keepdims=True)
            ao_h = jax.lax.dot_general(attn, vh, (((2,), (1,)), ((0,), (0,))),
                                       preferred_element_type=f32)
            ao_heads.append(ao_h)
        ao_c = jnp.concatenate(ao_heads, axis=2)
        ao_s[pl.ds(r0, CH)] = ao_c
        return dummy

    jax.lax.fori_loop(0, R // CH, att_chunk, 0)
    ao = ao_s[...]
    xo = res + (jnp.dot(ao.reshape(R * K, TOK), Wo_ref[...], preferred_element_type=f32)
                + bo_ref[...][None, :]).reshape(R, K, TOK)
    x2 = _ln(xo, ln2g_ref[...], ln2b_ref[...])
    ffh = jnp.dot(x2.reshape(R * K, TOK), Wf1_ref[...], preferred_element_type=f32) + bf1_ref[...][None, :]
    ffh = 0.5 * ffh * (1.0 + jax.lax.erf(ffh * (1.0 / math.sqrt(2.0))))
    ffo = jnp.dot(ffh, Wf2_ref[...], preferred_element_type=f32) + bf2_ref[...][None, :]
    xo = xo + ffo.reshape(R, K, TOK)
    z_int = jnp.mean(xo, axis=1)

    dh_in = jnp.concatenate([h_base, z_int], axis=1)
    dh = _ln(dh_in, lndg_ref[...], lndb_ref[...])
    d1 = jax.nn.relu(jnp.dot(dh, Wd1_ref[...], preferred_element_type=f32) + bd1_ref[...][None, :])
    delta = jnp.sum(d1 * Wd2_ref[...][:, 0][None, :], axis=1, keepdims=True) + bd2_ref[...][None, :]

    gm = jnp.mean(gate, axis=1, keepdims=True)
    gx = jnp.max(gate, axis=1, keepdims=True)
    gs = jnp.sqrt(jnp.mean((gate - gm) * (gate - gm), axis=1, keepdims=True))
    bh_in = jnp.concatenate([h_base, y_base, gm, gx, gs], axis=1)
    bh = _ln(bh_in, lnbg_ref[...], lnbb_ref[...])
    b1 = jax.nn.relu(jnp.dot(bh, Wbe1_ref[...], preferred_element_type=f32) + bbe1_ref[...][None, :])
    beta = jax.nn.sigmoid(jnp.sum(b1 * Wbe2_ref[...][:, 0][None, :], axis=1, keepdims=True)
                          + bbe2_ref[...][None, :])

    out_ref[...] = (y_base + beta * delta)[:, 0]


def kernel(x_num, W_gate, b_gate, value_proj, feature_embed, emb_tables, Wb1, bb1, Wb2, bb2, Wbase, bbase, ln1_g, ln1_b, Wq, bq, Wk, bk, Wv, bv, Wo, bo, ln2_g, ln2_b, Wf1, bf1, Wf2, bf2, lnd_g, lnd_b, Wd1, bd1, Wd2, bd2, lnb_g, lnb_b, Wbe1, bbe1, Wbe2, bbe2, x_cat):
    emb_bd = jnp.zeros((NCAT * CARD, NCAT * EMB), jnp.float32)
    for f in range(NCAT):
        emb_bd = emb_bd.at[f * CARD:(f + 1) * CARD, f * EMB:(f + 1) * EMB].set(emb_tables[f])
    VF = jnp.concatenate([value_proj, feature_embed], axis=1)
    Wqkv = jnp.concatenate([Wq, Wk, Wv], axis=1)
    bqkv = jnp.concatenate([bq, bk, bv], axis=0)
    ii = jnp.arange(2 * D)
    LT2 = ((ii[:, None] < ii[None, :]) & ((ii[:, None] // D) == (ii[None, :] // D))
           ).astype(jnp.bfloat16)
    x_cat = x_cat.astype(jnp.int32)

    n_tiles = B // R

    def full2(a):
        return pl.BlockSpec(a.shape, lambda i: (0, 0))

    def full1(a):
        return pl.BlockSpec(a.shape, lambda i: (0,))

    in_specs = [
        pl.BlockSpec((R, D), lambda i: (i, 0)),
        pl.BlockSpec((R, NCAT), lambda i: (i, 0)),
        full2(W_gate), full1(b_gate), full2(VF), full2(emb_bd), full2(LT2),
        full2(Wb1), full1(bb1), full2(Wb2), full1(bb2), full2(Wbase), full1(bbase),
        full1(ln1_g), full1(ln1_b), full2(Wqkv), full1(bqkv), full2(Wo), full1(bo),
        full1(ln2_g), full1(ln2_b), full2(Wf1), full1(bf1), full2(Wf2), full1(bf2),
        full1(lnd_g), full1(lnd_b), full2(Wd1), full1(bd1), full2(Wd2), full1(bd2),
        full1(lnb_g), full1(lnb_b), full2(Wbe1), full1(bbe1), full2(Wbe2), full1(bbe2),
    ]

    out = pl.pallas_call(
        _fwd_body,
        grid=(n_tiles,),
        in_specs=in_specs,
        out_specs=pl.BlockSpec((R,), lambda i: (i,)),
        out_shape=jax.ShapeDtypeStruct((B,), jnp.float32),
        compiler_params=pltpu.CompilerParams(
            dimension_semantics=("parallel",)),
        scratch_shapes=[
            pltpu.VMEM((R, K, TOK), jnp.float32),
            pltpu.VMEM((R, K, TOK), jnp.float32),
            pltpu.VMEM((R, K, TOK), jnp.float32),
            pltpu.VMEM((R, K, TOK), jnp.float32),
            pltpu.VMEM((R, K), jnp.float32),
        ],
    )(x_num, x_cat, W_gate, b_gate, VF, emb_bd, LT2,
      Wb1, bb1, Wb2, bb2, Wbase, bbase,
      ln1_g, ln1_b, Wqkv, bqkv, Wo, bo,
      ln2_g, ln2_b, Wf1, bf1, Wf2, bf2,
      lnd_g, lnd_b, Wd1, bd1, Wd2, bd2,
      lnb_g, lnb_b, Wbe1, bbe1, Wbe2, bbe2)
    return out
